# Initial kernel scaffold; baseline (speedup 1.0000x reference)
import jax
import jax.numpy as jnp
from jax.experimental import pallas as pl


def kernel(hidden_state, keys, values):
    raise NotImplementedError("write your pallas kernel here")


# speedup vs baseline: 1.2987x; 1.2987x over previous
import jax
import jax.numpy as jnp
from jax.experimental import pallas as pl
from jax.experimental.pallas import tpu as pltpu

D_MODEL = 128
BATCH = 32
TILE = 5000


def _attn_read_kernel(q_ref, k_ref, v_ref, o_ref, m_ref, l_ref, acc_ref):
    i = pl.program_id(0)
    n = pl.num_programs(0)

    @pl.when(i == 0)
    def _init():
        m_ref[...] = jnp.full(m_ref.shape, -jnp.inf, dtype=jnp.float32)
        l_ref[...] = jnp.zeros(l_ref.shape, dtype=jnp.float32)
        acc_ref[...] = jnp.zeros(acc_ref.shape, dtype=jnp.float32)

    q = q_ref[...]
    k = k_ref[...]
    s = jax.lax.dot_general(
        q, k, (((1,), (1,)), ((), ())), preferred_element_type=jnp.float32
    ) * (1.0 / (D_MODEL ** 0.5))

    m_prev = m_ref[...][:, 0:1]
    l_prev = l_ref[...][:, 0:1]
    m_cur = jnp.max(s, axis=1, keepdims=True)
    m_new = jnp.maximum(m_prev, m_cur)
    alpha = jnp.exp(m_prev - m_new)
    p = jnp.exp(s - m_new)
    l_new = alpha * l_prev + jnp.sum(p, axis=1, keepdims=True)

    pv = jax.lax.dot_general(
        p, v_ref[...], (((1,), (0,)), ((), ())), preferred_element_type=jnp.float32
    )
    acc_ref[...] = acc_ref[...] * alpha + pv
    m_ref[...] = jnp.broadcast_to(m_new, m_ref.shape)
    l_ref[...] = jnp.broadcast_to(l_new, l_ref.shape)

    @pl.when(i == n - 1)
    def _finish():
        o_ref[...] = acc_ref[...] / l_ref[...]


def kernel(hidden_state, keys, values):
    max_size = keys.shape[0]
    n_tiles = max_size // TILE
    return pl.pallas_call(
        _attn_read_kernel,
        grid=(n_tiles,),
        in_specs=[
            pl.BlockSpec((BATCH, D_MODEL), lambda i: (0, 0)),
            pl.BlockSpec((TILE, D_MODEL), lambda i: (i, 0)),
            pl.BlockSpec((TILE, D_MODEL), lambda i: (i, 0)),
        ],
        out_specs=pl.BlockSpec((BATCH, D_MODEL), lambda i: (0, 0)),
        out_shape=jax.ShapeDtypeStruct((BATCH, D_MODEL), jnp.float32),
        scratch_shapes=[
            pltpu.VMEM((BATCH, D_MODEL), jnp.float32),
            pltpu.VMEM((BATCH, D_MODEL), jnp.float32),
            pltpu.VMEM((BATCH, D_MODEL), jnp.float32),
        ],
    )(hidden_state, keys, values)


# speedup vs baseline: 1.4743x; 1.1352x over previous
import jax
import jax.numpy as jnp
from jax.experimental import pallas as pl
from jax.experimental.pallas import tpu as pltpu

D_MODEL = 128
BATCH = 32
TILE = 10000


def _attn_read_kernel(q_ref, k_ref, v_ref, o_ref, m_ref, l_ref, acc_ref):
    i = pl.program_id(0)
    n = pl.num_programs(0)

    @pl.when(i == 0)
    def _init():
        m_ref[...] = jnp.full(m_ref.shape, -jnp.inf, dtype=jnp.float32)
        l_ref[...] = jnp.zeros(l_ref.shape, dtype=jnp.float32)
        acc_ref[...] = jnp.zeros(acc_ref.shape, dtype=jnp.float32)

    q = q_ref[...]
    k = k_ref[...]
    s = jax.lax.dot_general(
        q, k, (((1,), (1,)), ((), ())), preferred_element_type=jnp.float32
    ) * (1.0 / (D_MODEL ** 0.5))

    m_prev = m_ref[...][:, 0:1]
    l_prev = l_ref[...][:, 0:1]
    m_cur = jnp.max(s, axis=1, keepdims=True)
    m_new = jnp.maximum(m_prev, m_cur)
    alpha = jnp.exp(m_prev - m_new)
    p = jnp.exp(s - m_new)
    l_new = alpha * l_prev + jnp.sum(p, axis=1, keepdims=True)

    pv = jax.lax.dot_general(
        p, v_ref[...], (((1,), (0,)), ((), ())), preferred_element_type=jnp.float32
    )
    acc_ref[...] = acc_ref[...] * alpha + pv
    m_ref[...] = jnp.broadcast_to(m_new, m_ref.shape)
    l_ref[...] = jnp.broadcast_to(l_new, l_ref.shape)

    @pl.when(i == n - 1)
    def _finish():
        o_ref[...] = acc_ref[...] / l_ref[...]


def kernel(hidden_state, keys, values):
    max_size = keys.shape[0]
    n_tiles = max_size // TILE
    return pl.pallas_call(
        _attn_read_kernel,
        grid=(n_tiles,),
        in_specs=[
            pl.BlockSpec((BATCH, D_MODEL), lambda i: (0, 0)),
            pl.BlockSpec((TILE, D_MODEL), lambda i: (i, 0)),
            pl.BlockSpec((TILE, D_MODEL), lambda i: (i, 0)),
        ],
        out_specs=pl.BlockSpec((BATCH, D_MODEL), lambda i: (0, 0)),
        out_shape=jax.ShapeDtypeStruct((BATCH, D_MODEL), jnp.float32),
        scratch_shapes=[
            pltpu.VMEM((BATCH, D_MODEL), jnp.float32),
            pltpu.VMEM((BATCH, D_MODEL), jnp.float32),
            pltpu.VMEM((BATCH, D_MODEL), jnp.float32),
        ],
    )(hidden_state, keys, values)
